# Initial kernel scaffold; baseline (speedup 1.0000x reference)
import jax
import jax.numpy as jnp
from jax.experimental import pallas as pl


def kernel(x, edge_index0, etypes0, edge_index1, etypes1, W1, ln1_w, ln1_b, loop_w1, bias1, W2, ln2_w, ln2_b, loop_w2, bias2, proj_w, proj_b):
    raise NotImplementedError("write your pallas kernel here")


# speedup vs baseline: 2.8937x; 2.8937x over previous
import functools

import jax
import jax.numpy as jnp
from jax import lax
from jax.experimental import pallas as pl
from jax.experimental.pallas import tpu as pltpu
from jax.experimental.pallas import tpu_sc as plsc

N = 10000
D = 128
E = 320000
NUM_RELS = 2

NTILES = 16
NPAD = 10240
TRASH = N
CH = 80
EPT = E // NTILES
NCHUNK = EPT // CH
ZB = 64
RPT = NPAD // NTILES


def _sc_body(x_hbm, src_hbm, dst_hbm, et_hbm, out_hbm,
             acc, src_v, dst_v, et_v, dst2_v, rows_v, zero_v, sem):
    rel = lax.axis_index("c")
    tid = lax.axis_index("s")

    zvec = jnp.zeros((16,), jnp.float32)

    def _zrow(i, c):
        for k in range(D // 16):
            zero_v[i, pl.ds(k * 16, 16)] = zvec
        return c

    lax.fori_loop(0, ZB, _zrow, 0)

    def _zcopy(j, c):
        pltpu.sync_copy(zero_v, acc.at[pl.ds(tid * RPT + j * ZB, ZB)])
        return c

    lax.fori_loop(0, RPT // ZB, _zcopy, 0)
    plsc.subcore_barrier()

    def _chunk(j, c):
        base = pl.multiple_of(tid * EPT + j * CH, 8)
        pltpu.sync_copy(src_hbm.at[pl.ds(base, CH)], src_v)
        pltpu.sync_copy(dst_hbm.at[pl.ds(base, CH)], dst_v)
        pltpu.sync_copy(et_hbm.at[pl.ds(base, CH)], et_v)
        for k in range(CH // 16):
            sl = pl.ds(k * 16, 16)
            dst2_v[sl] = jnp.where(et_v[sl] == rel, dst_v[sl], TRASH)
        pltpu.async_copy(x_hbm.at[src_v], rows_v, sem).wait()
        pltpu.sync_copy(rows_v, acc.at[dst2_v], add=True)
        return c

    lax.fori_loop(0, NCHUNK, _chunk, 0)
    plsc.subcore_barrier()

    pltpu.sync_copy(acc.at[pl.ds(tid * RPT, RPT)],
                    out_hbm.at[rel, pl.ds(tid * RPT, RPT)])


def _sc_segment(x_tab, src, dst, et):
    f = pl.kernel(
        _sc_body,
        out_type=jax.ShapeDtypeStruct((NUM_RELS, NPAD, D), jnp.float32),
        mesh=plsc.VectorSubcoreMesh(core_axis_name="c", subcore_axis_name="s"),
        scratch_types=[
            pltpu.VMEM_SHARED((NPAD, D), jnp.float32),
            pltpu.VMEM((CH,), jnp.int32),
            pltpu.VMEM((CH,), jnp.int32),
            pltpu.VMEM((CH,), jnp.int32),
            pltpu.VMEM((CH,), jnp.int32),
            pltpu.VMEM((CH, D), jnp.float32),
            pltpu.VMEM((ZB, D), jnp.float32),
            pltpu.SemaphoreType.DMA,
        ],
    )
    return f(x_tab, src, dst, et)


BN = 1000


def _tc_body(has_proj, a_ref, xin_ref, w0_ref, w1_ref, lw_ref, lnw_ref,
             lnb_ref, b_ref, *rest):
    if has_proj:
        pw_ref, pb_ref, o_ref = rest
    else:
        (o_ref,) = rest
    y = jnp.dot(a_ref[0], w0_ref[...], preferred_element_type=jnp.float32)
    y = y + jnp.dot(a_ref[1], w1_ref[...], preferred_element_type=jnp.float32)
    mu = jnp.mean(y, axis=-1, keepdims=True)
    var = jnp.mean((y - mu) ** 2, axis=-1, keepdims=True)
    h = (y - mu) * lax.rsqrt(var + 1e-5) * lnw_ref[...] + lnb_ref[...]
    h = h + jnp.dot(xin_ref[...], lw_ref[...],
                    preferred_element_type=jnp.float32) + b_ref[...]
    if has_proj:
        h = jnp.dot(h, pw_ref[...], preferred_element_type=jnp.float32) + pb_ref[...]
    o_ref[...] = h


def _tc_layer(A, x_in, w0, w1, loop_w, ln_w, ln_b, bias, pw=None, pb=None):
    has_proj = pw is not None
    full = lambda i: (0, 0)
    ins = [A, x_in, w0, w1, loop_w,
           ln_w.reshape(1, D), ln_b.reshape(1, D), bias.reshape(1, D)]
    specs = [
        pl.BlockSpec((NUM_RELS, BN, D), lambda i: (0, i, 0)),
        pl.BlockSpec((BN, D), lambda i: (i, 0)),
        pl.BlockSpec((D, D), full),
        pl.BlockSpec((D, D), full),
        pl.BlockSpec((D, D), full),
        pl.BlockSpec((1, D), full),
        pl.BlockSpec((1, D), full),
        pl.BlockSpec((1, D), full),
    ]
    if has_proj:
        ins += [pw, pb.reshape(1, D)]
        specs += [pl.BlockSpec((D, D), full), pl.BlockSpec((1, D), full)]
    return pl.pallas_call(
        functools.partial(_tc_body, has_proj),
        grid=(N // BN,),
        in_specs=specs,
        out_specs=pl.BlockSpec((BN, D), lambda i: (i, 0)),
        out_shape=jax.ShapeDtypeStruct((N, D), jnp.float32),
    )(*ins)


def kernel(x, edge_index0, etypes0, edge_index1, etypes1, W1, ln1_w, ln1_b,
           loop_w1, bias1, W2, ln2_w, ln2_b, loop_w2, bias2, proj_w, proj_b):
    A1 = _sc_segment(x, edge_index0[0], edge_index0[1], etypes0)
    h1 = _tc_layer(A1, x, W1[0], W1[1], loop_w1, ln1_w, ln1_b, bias1)
    A2 = _sc_segment(h1, edge_index1[0], edge_index1[1], etypes1)
    emb = _tc_layer(A2, h1, W2[0], W2[1], loop_w2, ln2_w, ln2_b, bias2,
                    proj_w, proj_b)
    return emb


# speedup vs baseline: 7.6576x; 2.6463x over previous
import functools

import jax
import jax.numpy as jnp
from jax import lax
from jax.experimental import pallas as pl
from jax.experimental.pallas import tpu as pltpu
from jax.experimental.pallas import tpu_sc as plsc

N = 10000
D = 128
E = 320000
NUM_RELS = 2

NTILES = 16
NPAD = 10240
TRASH = N
CH = 80
EPT = E // NTILES
NCHUNK = EPT // CH
NBUF = 3
ZB = 16
RPT = NPAD // NTILES


def _sc_body(x_hbm, src_hbm, dst_hbm, et_hbm, out_hbm,
             acc, srcb, dstb, etb, dst2, rows, zero_v, *sems):
    esem = sems[:NBUF]
    gsem = sems[NBUF:]
    rel = lax.axis_index("c")
    tid = lax.axis_index("s")

    zvec = jnp.zeros((16,), jnp.float32)

    def _zrow(i, c):
        for k in range(D // 16):
            zero_v[i, pl.ds(k * 16, 16)] = zvec
        return c

    lax.fori_loop(0, ZB, _zrow, 0)

    def _zcopy(j, c):
        pltpu.sync_copy(zero_v, acc.at[pl.ds(tid * RPT + j * ZB, ZB)])
        return c

    lax.fori_loop(0, RPT // ZB, _zcopy, 0)
    plsc.subcore_barrier()

    def _lstart(k, b):
        base = pl.multiple_of(tid * EPT, 8) + k * CH
        pltpu.async_copy(src_hbm.at[pl.ds(base, CH)], srcb.at[b], esem[b])
        pltpu.async_copy(dst_hbm.at[pl.ds(base, CH)], dstb.at[b], esem[b])
        pltpu.async_copy(et_hbm.at[pl.ds(base, CH)], etb.at[b], esem[b])

    def _lwait(b):
        pltpu.make_async_copy(src_hbm.at[pl.ds(0, CH)], srcb.at[b],
                              esem[b]).wait()
        pltpu.make_async_copy(dst_hbm.at[pl.ds(0, CH)], dstb.at[b],
                              esem[b]).wait()
        pltpu.make_async_copy(et_hbm.at[pl.ds(0, CH)], etb.at[b],
                              esem[b]).wait()

    def _compute(b):
        for k in range(CH // 16):
            sl = pl.ds(k * 16, 16)
            dst2[b, sl] = jnp.where(etb[b, sl] == rel, dstb[b, sl], TRASH)

    def _gstart(b):
        pltpu.async_copy(x_hbm.at[srcb.at[b]], rows.at[b], gsem[b])

    def _gwait(b):
        pltpu.make_async_copy(x_hbm.at[srcb.at[0]], rows.at[b],
                              gsem[b]).wait()

    def _scat(b):
        pltpu.sync_copy(rows.at[b], acc.at[dst2.at[b]], add=True)

    def _iter(k, b, b2, do_pre, do_l):
        if do_pre:
            _lwait(b2)
            _compute(b2)
            _gstart(b2)
        _gwait(b)
        if do_l:
            _lstart(k + 3, b)
        _scat(b)

    for kk in range(NBUF):
        _lstart(kk, kk)
    for kk in range(2):
        _lwait(kk)
        _compute(kk)
        _gstart(kk)

    NSTEADY = 246

    def _outer(i, c):
        for b in range(NBUF):
            k = i * NBUF + b
            _iter(k, b, (b + 2) % NBUF, True, True)
        return c

    lax.fori_loop(0, NSTEADY // NBUF, _outer, 0)
    for k in range(NSTEADY, NCHUNK):
        _iter(k, k % NBUF, (k + 2) % NBUF,
              do_pre=(k + 2 < NCHUNK), do_l=(k + 3 < NCHUNK))

    plsc.subcore_barrier()

    pltpu.sync_copy(acc.at[pl.ds(tid * RPT, RPT)],
                    out_hbm.at[rel, pl.ds(tid * RPT, RPT)])


def _sc_segment(x_tab, src, dst, et):
    f = pl.kernel(
        _sc_body,
        out_type=jax.ShapeDtypeStruct((NUM_RELS, NPAD, D), jnp.float32),
        mesh=plsc.VectorSubcoreMesh(core_axis_name="c", subcore_axis_name="s"),
        scratch_types=[
            pltpu.VMEM_SHARED((NPAD, D), jnp.float32),
            pltpu.VMEM((NBUF, CH), jnp.int32),
            pltpu.VMEM((NBUF, CH), jnp.int32),
            pltpu.VMEM((NBUF, CH), jnp.int32),
            pltpu.VMEM((NBUF, CH), jnp.int32),
            pltpu.VMEM((NBUF, CH, D), jnp.float32),
            pltpu.VMEM((ZB, D), jnp.float32),
        ] + [pltpu.SemaphoreType.DMA] * (2 * NBUF),
    )
    return f(x_tab, src, dst, et)


BN = 1000


def _tc_body(has_proj, a_ref, xin_ref, w0_ref, w1_ref, lw_ref, lnw_ref,
             lnb_ref, b_ref, *rest):
    if has_proj:
        pw_ref, pb_ref, o_ref = rest
    else:
        (o_ref,) = rest
    y = jnp.dot(a_ref[0], w0_ref[...], preferred_element_type=jnp.float32)
    y = y + jnp.dot(a_ref[1], w1_ref[...], preferred_element_type=jnp.float32)
    mu = jnp.mean(y, axis=-1, keepdims=True)
    var = jnp.mean((y - mu) ** 2, axis=-1, keepdims=True)
    h = (y - mu) * lax.rsqrt(var + 1e-5) * lnw_ref[...] + lnb_ref[...]
    h = h + jnp.dot(xin_ref[...], lw_ref[...],
                    preferred_element_type=jnp.float32) + b_ref[...]
    if has_proj:
        h = jnp.dot(h, pw_ref[...], preferred_element_type=jnp.float32) + pb_ref[...]
    o_ref[...] = h


def _tc_layer(A, x_in, w0, w1, loop_w, ln_w, ln_b, bias, pw=None, pb=None):
    has_proj = pw is not None
    full = lambda i: (0, 0)
    ins = [A, x_in, w0, w1, loop_w,
           ln_w.reshape(1, D), ln_b.reshape(1, D), bias.reshape(1, D)]
    specs = [
        pl.BlockSpec((NUM_RELS, BN, D), lambda i: (0, i, 0)),
        pl.BlockSpec((BN, D), lambda i: (i, 0)),
        pl.BlockSpec((D, D), full),
        pl.BlockSpec((D, D), full),
        pl.BlockSpec((D, D), full),
        pl.BlockSpec((1, D), full),
        pl.BlockSpec((1, D), full),
        pl.BlockSpec((1, D), full),
    ]
    if has_proj:
        ins += [pw, pb.reshape(1, D)]
        specs += [pl.BlockSpec((D, D), full), pl.BlockSpec((1, D), full)]
    return pl.pallas_call(
        functools.partial(_tc_body, has_proj),
        grid=(N // BN,),
        in_specs=specs,
        out_specs=pl.BlockSpec((BN, D), lambda i: (i, 0)),
        out_shape=jax.ShapeDtypeStruct((N, D), jnp.float32),
    )(*ins)


def kernel(x, edge_index0, etypes0, edge_index1, etypes1, W1, ln1_w, ln1_b,
           loop_w1, bias1, W2, ln2_w, ln2_b, loop_w2, bias2, proj_w, proj_b):
    A1 = _sc_segment(x, edge_index0[0], edge_index0[1], etypes0)
    h1 = _tc_layer(A1, x, W1[0], W1[1], loop_w1, ln1_w, ln1_b, bias1)
    A2 = _sc_segment(h1, edge_index1[0], edge_index1[1], etypes1)
    emb = _tc_layer(A2, h1, W2[0], W2[1], loop_w2, ln2_w, ln2_b, bias2,
                    proj_w, proj_b)
    return emb


# speedup vs baseline: 10.0033x; 1.3063x over previous
import functools

import jax
import jax.numpy as jnp
from jax import lax
from jax.experimental import pallas as pl
from jax.experimental.pallas import tpu as pltpu
from jax.experimental.pallas import tpu_sc as plsc

N = 10000
D = 128
E = 320000
NUM_RELS = 2

NTILES = 16
NPAD = 10240
TRASH = N
CH = 80
EPT = E // NTILES
NCHUNK = EPT // CH
NBUF = 3
ZB = 16
RPT = NPAD // NTILES


def _sc_body(x_hbm, src_hbm, dst_hbm, et_hbm, out_hbm,
             acc, srcb, dstb, etb, dst2, rows, zero_v, *sems):
    esem = sems[:NBUF]
    gsem = sems[NBUF:]
    rel = lax.axis_index("c")
    tid = lax.axis_index("s")

    zvec = jnp.zeros((16,), jnp.float32)

    def _zrow(i, c):
        for k in range(D // 16):
            zero_v[i, pl.ds(k * 16, 16)] = zvec
        return c

    lax.fori_loop(0, ZB, _zrow, 0)

    def _zcopy(j, c):
        pltpu.sync_copy(zero_v, acc.at[pl.ds(tid * RPT + j * ZB, ZB)])
        return c

    lax.fori_loop(0, RPT // ZB, _zcopy, 0)
    plsc.subcore_barrier()

    def _lstart(k, b):
        base = pl.multiple_of(tid * EPT, 8) + k * CH
        pltpu.async_copy(src_hbm.at[pl.ds(base, CH)], srcb.at[b], esem[b])
        pltpu.async_copy(dst_hbm.at[pl.ds(base, CH)], dstb.at[b], esem[b])
        pltpu.async_copy(et_hbm.at[pl.ds(base, CH)], etb.at[b], esem[b])

    def _lwait(b):
        pltpu.make_async_copy(src_hbm.at[pl.ds(0, CH)], srcb.at[b],
                              esem[b]).wait()
        pltpu.make_async_copy(dst_hbm.at[pl.ds(0, CH)], dstb.at[b],
                              esem[b]).wait()
        pltpu.make_async_copy(et_hbm.at[pl.ds(0, CH)], etb.at[b],
                              esem[b]).wait()

    trash = TRASH + tid

    def _compute(b):
        for k in range(CH // 16):
            sl = pl.ds(k * 16, 16)
            dst2[b, sl] = jnp.where(etb[b, sl] == rel, dstb[b, sl], trash)

    def _gstart(b):
        pltpu.async_copy(x_hbm.at[srcb.at[b]], rows.at[b], gsem[b])

    def _gwait(b):
        pltpu.make_async_copy(x_hbm.at[srcb.at[0]], rows.at[b],
                              gsem[b]).wait()

    def _scat(b):
        pltpu.sync_copy(rows.at[b], acc.at[dst2.at[b]], add=True)

    def _iter(k, b, b2, do_pre, do_l):
        if do_pre:
            _lwait(b2)
            _compute(b2)
            _gstart(b2)
        _gwait(b)
        if do_l:
            _lstart(k + 3, b)
        _scat(b)

    for kk in range(NBUF):
        _lstart(kk, kk)
    for kk in range(2):
        _lwait(kk)
        _compute(kk)
        _gstart(kk)

    NSTEADY = 246

    def _outer(i, c):
        for b in range(NBUF):
            k = i * NBUF + b
            _iter(k, b, (b + 2) % NBUF, True, True)
        return c

    lax.fori_loop(0, NSTEADY // NBUF, _outer, 0)
    for k in range(NSTEADY, NCHUNK):
        _iter(k, k % NBUF, (k + 2) % NBUF,
              do_pre=(k + 2 < NCHUNK), do_l=(k + 3 < NCHUNK))

    plsc.subcore_barrier()

    pltpu.sync_copy(acc.at[pl.ds(tid * RPT, RPT)],
                    out_hbm.at[rel, pl.ds(tid * RPT, RPT)])


def _sc_segment(x_tab, src, dst, et):
    f = pl.kernel(
        _sc_body,
        out_type=jax.ShapeDtypeStruct((NUM_RELS, NPAD, D), jnp.float32),
        mesh=plsc.VectorSubcoreMesh(core_axis_name="c", subcore_axis_name="s"),
        scratch_types=[
            pltpu.VMEM_SHARED((NPAD, D), jnp.float32),
            pltpu.VMEM((NBUF, CH), jnp.int32),
            pltpu.VMEM((NBUF, CH), jnp.int32),
            pltpu.VMEM((NBUF, CH), jnp.int32),
            pltpu.VMEM((NBUF, CH), jnp.int32),
            pltpu.VMEM((NBUF, CH, D), jnp.float32),
            pltpu.VMEM((ZB, D), jnp.float32),
        ] + [pltpu.SemaphoreType.DMA] * (2 * NBUF),
    )
    return f(x_tab, src, dst, et)


BN = 1000


def _tc_body(has_proj, a_ref, xin_ref, w0_ref, w1_ref, lw_ref, lnw_ref,
             lnb_ref, b_ref, *rest):
    if has_proj:
        pw_ref, pb_ref, o_ref = rest
    else:
        (o_ref,) = rest
    y = jnp.dot(a_ref[0], w0_ref[...], preferred_element_type=jnp.float32)
    y = y + jnp.dot(a_ref[1], w1_ref[...], preferred_element_type=jnp.float32)
    mu = jnp.mean(y, axis=-1, keepdims=True)
    var = jnp.mean((y - mu) ** 2, axis=-1, keepdims=True)
    h = (y - mu) * lax.rsqrt(var + 1e-5) * lnw_ref[...] + lnb_ref[...]
    h = h + jnp.dot(xin_ref[...], lw_ref[...],
                    preferred_element_type=jnp.float32) + b_ref[...]
    if has_proj:
        h = jnp.dot(h, pw_ref[...], preferred_element_type=jnp.float32) + pb_ref[...]
    o_ref[...] = h


def _tc_layer(A, x_in, w0, w1, loop_w, ln_w, ln_b, bias, pw=None, pb=None):
    has_proj = pw is not None
    full = lambda i: (0, 0)
    ins = [A, x_in, w0, w1, loop_w,
           ln_w.reshape(1, D), ln_b.reshape(1, D), bias.reshape(1, D)]
    specs = [
        pl.BlockSpec((NUM_RELS, BN, D), lambda i: (0, i, 0)),
        pl.BlockSpec((BN, D), lambda i: (i, 0)),
        pl.BlockSpec((D, D), full),
        pl.BlockSpec((D, D), full),
        pl.BlockSpec((D, D), full),
        pl.BlockSpec((1, D), full),
        pl.BlockSpec((1, D), full),
        pl.BlockSpec((1, D), full),
    ]
    if has_proj:
        ins += [pw, pb.reshape(1, D)]
        specs += [pl.BlockSpec((D, D), full), pl.BlockSpec((1, D), full)]
    return pl.pallas_call(
        functools.partial(_tc_body, has_proj),
        grid=(N // BN,),
        in_specs=specs,
        out_specs=pl.BlockSpec((BN, D), lambda i: (i, 0)),
        out_shape=jax.ShapeDtypeStruct((N, D), jnp.float32),
    )(*ins)


def kernel(x, edge_index0, etypes0, edge_index1, etypes1, W1, ln1_w, ln1_b,
           loop_w1, bias1, W2, ln2_w, ln2_b, loop_w2, bias2, proj_w, proj_b):
    A1 = _sc_segment(x, edge_index0[0], edge_index0[1], etypes0)
    h1 = _tc_layer(A1, x, W1[0], W1[1], loop_w1, ln1_w, ln1_b, bias1)
    A2 = _sc_segment(h1, edge_index1[0], edge_index1[1], etypes1)
    emb = _tc_layer(A2, h1, W2[0], W2[1], loop_w2, ln2_w, ln2_b, bias2,
                    proj_w, proj_b)
    return emb


# speedup vs baseline: 10.0203x; 1.0017x over previous
import functools

import jax
import jax.numpy as jnp
from jax import lax
from jax.experimental import pallas as pl
from jax.experimental.pallas import tpu as pltpu
from jax.experimental.pallas import tpu_sc as plsc

N = 10000
D = 128
E = 320000
NUM_RELS = 2

NTILES = 16
NPAD = 10240
TRASH = N
CH = 80
EPT = E // NTILES
NCHUNK = EPT // CH
NBUF = 4
ZB = 16
RPT = NPAD // NTILES


def _sc_body(x_hbm, src_hbm, dst_hbm, et_hbm, out_hbm,
             acc, srcb, dstb, etb, dst2, rows, zero_v, *sems):
    esem = sems[:NBUF]
    gsem = sems[NBUF:2 * NBUF]
    ssem = sems[2 * NBUF:]
    rel = lax.axis_index("c")
    tid = lax.axis_index("s")

    zvec = jnp.zeros((16,), jnp.float32)

    def _zrow(i, c):
        for k in range(D // 16):
            zero_v[i, pl.ds(k * 16, 16)] = zvec
        return c

    lax.fori_loop(0, ZB, _zrow, 0)

    def _zcopy(j, c):
        pltpu.sync_copy(zero_v, acc.at[pl.ds(tid * RPT + j * ZB, ZB)])
        return c

    lax.fori_loop(0, RPT // ZB, _zcopy, 0)
    plsc.subcore_barrier()

    def _lstart(k, b):
        base = pl.multiple_of(tid * EPT, 8) + k * CH
        pltpu.async_copy(src_hbm.at[pl.ds(base, CH)], srcb.at[b], esem[b])
        pltpu.async_copy(dst_hbm.at[pl.ds(base, CH)], dstb.at[b], esem[b])
        pltpu.async_copy(et_hbm.at[pl.ds(base, CH)], etb.at[b], esem[b])

    def _lwait(b):
        pltpu.make_async_copy(src_hbm.at[pl.ds(0, CH)], srcb.at[b],
                              esem[b]).wait()
        pltpu.make_async_copy(dst_hbm.at[pl.ds(0, CH)], dstb.at[b],
                              esem[b]).wait()
        pltpu.make_async_copy(et_hbm.at[pl.ds(0, CH)], etb.at[b],
                              esem[b]).wait()

    trash = TRASH + tid

    def _compute(b):
        for k in range(CH // 16):
            sl = pl.ds(k * 16, 16)
            dst2[b, sl] = jnp.where(etb[b, sl] == rel, dstb[b, sl], trash)

    def _gstart(b):
        pltpu.async_copy(x_hbm.at[srcb.at[b]], rows.at[b], gsem[b])

    def _gwait(b):
        pltpu.make_async_copy(x_hbm.at[srcb.at[0]], rows.at[b],
                              gsem[b]).wait()

    def _sstart(b):
        pltpu.async_copy(rows.at[b], acc.at[dst2.at[b]], ssem[b], add=True)

    def _swait(b):
        pltpu.make_async_copy(rows.at[0], acc.at[dst2.at[0]],
                              ssem[b]).wait()

    def _iter(k, b, b2, do_pre, do_swait, do_l):
        if do_pre:
            if do_swait:
                _swait(b2)
            _lwait(b2)
            _compute(b2)
            _gstart(b2)
        _gwait(b)
        if do_l:
            _lstart(k + 4, b)
        _sstart(b)

    for kk in range(NBUF):
        _lstart(kk, kk)
    for kk in range(2):
        _lwait(kk)
        _compute(kk)
        _gstart(kk)

    for k in range(6):
        _iter(k, k % NBUF, (k + 2) % NBUF,
              do_pre=True, do_swait=(k >= 2), do_l=True)

    NSTEADY = 246

    def _outer(i, c):
        for j in range(NBUF):
            k = 6 + i * NBUF + j
            _iter(k, (6 + j) % NBUF, (6 + j + 2) % NBUF,
                  do_pre=True, do_swait=True, do_l=True)
        return c

    lax.fori_loop(0, (NSTEADY - 6) // NBUF, _outer, 0)
    for k in range(NSTEADY, NCHUNK):
        _iter(k, k % NBUF, (k + 2) % NBUF,
              do_pre=(k + 2 < NCHUNK), do_swait=(k + 2 < NCHUNK),
              do_l=(k + 4 < NCHUNK))

    for b in range(NBUF):
        _swait(b)

    plsc.subcore_barrier()

    pltpu.sync_copy(acc.at[pl.ds(tid * RPT, RPT)],
                    out_hbm.at[rel, pl.ds(tid * RPT, RPT)])


def _sc_segment(x_tab, src, dst, et):
    f = pl.kernel(
        _sc_body,
        out_type=jax.ShapeDtypeStruct((NUM_RELS, NPAD, D), jnp.float32),
        mesh=plsc.VectorSubcoreMesh(core_axis_name="c", subcore_axis_name="s"),
        scratch_types=[
            pltpu.VMEM_SHARED((NPAD, D), jnp.float32),
            pltpu.VMEM((NBUF, CH), jnp.int32),
            pltpu.VMEM((NBUF, CH), jnp.int32),
            pltpu.VMEM((NBUF, CH), jnp.int32),
            pltpu.VMEM((NBUF, CH), jnp.int32),
            pltpu.VMEM((NBUF, CH, D), jnp.float32),
            pltpu.VMEM((ZB, D), jnp.float32),
        ] + [pltpu.SemaphoreType.DMA] * (3 * NBUF),
    )
    return f(x_tab, src, dst, et)


BN = 1000


def _tc_body(has_proj, a_ref, xin_ref, w0_ref, w1_ref, lw_ref, lnw_ref,
             lnb_ref, b_ref, *rest):
    if has_proj:
        pw_ref, pb_ref, o_ref = rest
    else:
        (o_ref,) = rest
    y = jnp.dot(a_ref[0], w0_ref[...], preferred_element_type=jnp.float32)
    y = y + jnp.dot(a_ref[1], w1_ref[...], preferred_element_type=jnp.float32)
    mu = jnp.mean(y, axis=-1, keepdims=True)
    var = jnp.mean((y - mu) ** 2, axis=-1, keepdims=True)
    h = (y - mu) * lax.rsqrt(var + 1e-5) * lnw_ref[...] + lnb_ref[...]
    h = h + jnp.dot(xin_ref[...], lw_ref[...],
                    preferred_element_type=jnp.float32) + b_ref[...]
    if has_proj:
        h = jnp.dot(h, pw_ref[...], preferred_element_type=jnp.float32) + pb_ref[...]
    o_ref[...] = h


def _tc_layer(A, x_in, w0, w1, loop_w, ln_w, ln_b, bias, pw=None, pb=None):
    has_proj = pw is not None
    full = lambda i: (0, 0)
    ins = [A, x_in, w0, w1, loop_w,
           ln_w.reshape(1, D), ln_b.reshape(1, D), bias.reshape(1, D)]
    specs = [
        pl.BlockSpec((NUM_RELS, BN, D), lambda i: (0, i, 0)),
        pl.BlockSpec((BN, D), lambda i: (i, 0)),
        pl.BlockSpec((D, D), full),
        pl.BlockSpec((D, D), full),
        pl.BlockSpec((D, D), full),
        pl.BlockSpec((1, D), full),
        pl.BlockSpec((1, D), full),
        pl.BlockSpec((1, D), full),
    ]
    if has_proj:
        ins += [pw, pb.reshape(1, D)]
        specs += [pl.BlockSpec((D, D), full), pl.BlockSpec((1, D), full)]
    return pl.pallas_call(
        functools.partial(_tc_body, has_proj),
        grid=(N // BN,),
        in_specs=specs,
        out_specs=pl.BlockSpec((BN, D), lambda i: (i, 0)),
        out_shape=jax.ShapeDtypeStruct((N, D), jnp.float32),
    )(*ins)


def kernel(x, edge_index0, etypes0, edge_index1, etypes1, W1, ln1_w, ln1_b,
           loop_w1, bias1, W2, ln2_w, ln2_b, loop_w2, bias2, proj_w, proj_b):
    A1 = _sc_segment(x, edge_index0[0], edge_index0[1], etypes0)
    h1 = _tc_layer(A1, x, W1[0], W1[1], loop_w1, ln1_w, ln1_b, bias1)
    A2 = _sc_segment(h1, edge_index1[0], edge_index1[1], etypes1)
    emb = _tc_layer(A2, h1, W2[0], W2[1], loop_w2, ln2_w, ln2_b, bias2,
                    proj_w, proj_b)
    return emb
